# Initial kernel scaffold; baseline (speedup 1.0000x reference)
import functools

import jax
import jax.numpy as jnp
from jax import lax
from jax.experimental import pallas as pl
from jax.experimental.pallas import tpu as pltpu
from jax.experimental.pallas import tpu_sc as plsc

NC = 2
NS = 16
CH = 80


def _segsum_kernel(n_nodes, d, rows_per_tile, src_base, dst_base):
    npt = n_nodes // NS
    mesh = plsc.VectorSubcoreMesh(core_axis_name="c", subcore_axis_name="s")

    @functools.partial(
        pl.kernel,
        out_type=jax.ShapeDtypeStruct((2 * n_nodes, d), jnp.float32),
        mesh=mesh,
        scratch_types=[
            pltpu.VMEM((rows_per_tile, CH), jnp.int32),
            pltpu.VMEM((rows_per_tile, CH), jnp.int32),
            pltpu.VMEM((CH, d), jnp.float32),
            pltpu.VMEM_SHARED((n_nodes, d), jnp.float32),
        ],
    )
    def seg(table_hbm, src2_hbm, dst2_hbm, zeros_hbm, out_hbm,
            src_v, dst_v, rows_v, accum):
        cid = lax.axis_index("c")
        sid = lax.axis_index("s")
        pltpu.sync_copy(zeros_hbm, accum.at[pl.ds(sid * npt, npt)])
        pltpu.sync_copy(src2_hbm.at[pl.ds(src_base(cid, sid), rows_per_tile)],
                        src_v)
        pltpu.sync_copy(dst2_hbm.at[pl.ds(dst_base(cid, sid), rows_per_tile)],
                        dst_v)
        plsc.subcore_barrier()

        def body(g, carry):
            pltpu.sync_copy(table_hbm.at[src_v.at[g]], rows_v)
            pltpu.sync_copy(rows_v, accum.at[dst_v.at[g]], add=True)
            return carry

        lax.fori_loop(0, rows_per_tile, body, 0)
        plsc.subcore_barrier()
        pltpu.sync_copy(accum.at[pl.ds(sid * npt, npt)],
                        out_hbm.at[pl.ds(cid * n_nodes + sid * npt, npt)])

    return seg


def _tc_gates(ax_ref, ah_ref, h_ref, wxru_ref, whru_ref, bru_ref, wcx_ref,
              hn_ref, u_ref, p_ref):
    ax = ax_ref[...]
    z = jnp.dot(ax, wxru_ref[...], preferred_element_type=jnp.float32)
    z = z + jnp.dot(ah_ref[...], whru_ref[...],
                    preferred_element_type=jnp.float32)
    ru = jax.nn.sigmoid(z + bru_ref[...])
    d = ax.shape[1]
    r = ru[:, :d]
    hn_ref[...] = r * h_ref[...]
    u_ref[...] = ru[:, d:]
    p_ref[...] = jnp.dot(ax, wcx_ref[...], preferred_element_type=jnp.float32)


def _tc_out(p_ref, b0_ref, b1_ref, h_ref, u_ref, wch_ref, bc_ref, out_ref):
    b = b0_ref[...] + b1_ref[...]
    c = jax.nn.sigmoid(p_ref[...]
                       + jnp.dot(b, wch_ref[...],
                                 preferred_element_type=jnp.float32)
                       + bc_ref[...])
    u = u_ref[...]
    out_ref[...] = u * h_ref[...] + (1.0 - u) * c


def kernel(x, h, edge_index, W_r, b_r, W_u, b_u, W_c, b_c,
           r_bias, u_bias, c_bias):
    n, d_in = x.shape
    d_out = h.shape[1]
    e = edge_index.shape[1]
    src = edge_index[0]
    dst = edge_index[1]

    xh2 = jnp.concatenate([x, h], axis=0)
    srcs2 = jnp.concatenate([src, src + n]).reshape(-1, CH)
    dst2 = dst.reshape(-1, CH)
    zeros = jnp.zeros((n // NS, d_out), jnp.float32)

    rpt1 = e // CH // NS
    seg1 = _segsum_kernel(
        n, d_out, rpt1,
        src_base=lambda cid, sid: cid * (e // CH) + sid * rpt1,
        dst_base=lambda cid, sid: sid * rpt1,
    )
    agg = seg1(xh2, srcs2, dst2, zeros)
    a_x, a_h = agg[:n], agg[n:]

    w_ru = jnp.concatenate([W_r, W_u], axis=1)
    wx_ru = w_ru[:d_in]
    wh_ru = w_ru[d_in:]
    b_ru = jnp.concatenate([b_r + r_bias, b_u + u_bias]).reshape(1, -1)
    wc_x = W_c[:d_in]
    wc_h = W_c[d_in:]
    b_cc = (b_c + c_bias).reshape(1, -1)

    br = 2000
    grid = (n // br,)
    row_spec = pl.BlockSpec((br, d_out), lambda i: (i, 0))
    full = lambda s: pl.BlockSpec(s, lambda i: (0, 0))
    h_, u, p = pl.pallas_call(
        _tc_gates,
        grid=grid,
        in_specs=[row_spec, row_spec, row_spec,
                  full((d_in, 2 * d_out)), full((d_out, 2 * d_out)),
                  full((1, 2 * d_out)), full((d_in, d_out))],
        out_specs=[row_spec, row_spec, row_spec],
        out_shape=[jax.ShapeDtypeStruct((n, d_out), jnp.float32)] * 3,
    )(a_x, a_h, h, wx_ru, wh_ru, b_ru, wc_x)

    src2b = src.reshape(-1, CH)
    rpt2 = e // CH // (NC * NS)
    seg2 = _segsum_kernel(
        n, d_out, rpt2,
        src_base=lambda cid, sid: (cid * NS + sid) * rpt2,
        dst_base=lambda cid, sid: (cid * NS + sid) * rpt2,
    )
    part = seg2(h_, src2b, dst2, zeros)
    b0, b1 = part[:n], part[n:]

    new_h = pl.pallas_call(
        _tc_out,
        grid=grid,
        in_specs=[row_spec, row_spec, row_spec, row_spec, row_spec,
                  full((d_out, d_out)), full((1, d_out))],
        out_specs=row_spec,
        out_shape=jax.ShapeDtypeStruct((n, d_out), jnp.float32),
    )(p, b0, b1, h, u, wc_h, b_cc)
    return new_h


# speedup vs baseline: 7.6085x; 7.6085x over previous
import functools

import jax
import jax.numpy as jnp
from jax import lax
from jax.experimental import pallas as pl
from jax.experimental.pallas import tpu as pltpu
from jax.experimental.pallas import tpu_sc as plsc

NC = 2
NS = 16
CH = 128


def _segsum_kernel(np_rows, d, rows_per_tile, src_base, dst_base):
    npt = np_rows // NS
    mesh = plsc.VectorSubcoreMesh(core_axis_name="c", subcore_axis_name="s")

    @functools.partial(
        pl.kernel,
        out_type=jax.ShapeDtypeStruct((2 * np_rows, d), jnp.float32),
        mesh=mesh,
        scratch_types=[
            pltpu.VMEM((8, CH), jnp.int32),
            pltpu.VMEM((8, CH), jnp.int32),
            pltpu.VMEM((CH, d), jnp.float32),
            pltpu.VMEM_SHARED((np_rows, d), jnp.float32),
        ],
    )
    def seg(table_hbm, src2_hbm, dst2_hbm, zeros_hbm, out_hbm,
            src_v, dst_v, rows_v, accum):
        cid = lax.axis_index("c")
        sid = lax.axis_index("s")
        pltpu.sync_copy(zeros_hbm, accum.at[pl.ds(sid * npt, npt)])
        plsc.subcore_barrier()

        def body(j, carry):
            pltpu.sync_copy(
                src2_hbm.at[pl.ds(src_base(cid, sid) + j * 8, 8)], src_v)
            pltpu.sync_copy(
                dst2_hbm.at[pl.ds(dst_base(cid, sid) + j * 8, 8)], dst_v)
            for g in range(8):
                pltpu.sync_copy(table_hbm.at[src_v.at[g]], rows_v)
                pltpu.sync_copy(rows_v, accum.at[dst_v.at[g]], add=True)
            return carry

        lax.fori_loop(0, rows_per_tile // 8, body, 0)
        plsc.subcore_barrier()
        pltpu.sync_copy(accum.at[pl.ds(sid * npt, npt)],
                        out_hbm.at[pl.ds(cid * np_rows + sid * npt, npt)])

    return seg


def _tc_gates(ax_ref, ah_ref, h_ref, wxru_ref, whru_ref, bru_ref, wcx_ref,
              hna_ref, u_ref, p_ref):
    ax = ax_ref[...]
    z = jnp.dot(ax, wxru_ref[...], preferred_element_type=jnp.float32)
    z = z + jnp.dot(ah_ref[...], whru_ref[...],
                    preferred_element_type=jnp.float32)
    ru = jax.nn.sigmoid(z + bru_ref[...])
    d = ax.shape[1]
    hna_ref[...] = ru[:, :d] * h_ref[...]
    u_ref[...] = ru[:, d:]
    p_ref[...] = jnp.dot(ax, wcx_ref[...], preferred_element_type=jnp.float32)


def _tc_out(p_ref, b0_ref, b1_ref, h_ref, u_ref, wch_ref, bc_ref, out_ref):
    b = b0_ref[...] + b1_ref[...]
    c = jax.nn.sigmoid(p_ref[...]
                       + jnp.dot(b, wch_ref[...],
                                 preferred_element_type=jnp.float32)
                       + bc_ref[...])
    u = u_ref[...]
    out_ref[...] = u * h_ref[...] + (1.0 - u) * c


def kernel(x, h, edge_index, W_r, b_r, W_u, b_u, W_c, b_c,
           r_bias, u_bias, c_bias):
    n, d_in = x.shape
    d_out = h.shape[1]
    e = edge_index.shape[1]
    src = edge_index[0]
    dst = edge_index[1]

    quant = CH * NS * NC * 8
    e_pad = -(-e // quant) * quant
    npad = e_pad - e
    np_rows = -(-n // (NS * 8)) * (NS * 8)
    pad_src = jnp.arange(npad, dtype=jnp.int32) % n
    pad_dst = n + jnp.arange(npad, dtype=jnp.int32) % (np_rows - n)

    xh2 = jnp.concatenate([x, h], axis=0)
    srcs1 = jnp.concatenate(
        [src, pad_src, src + n, pad_src + n]).reshape(-1, CH)
    dst2 = jnp.concatenate([dst, pad_dst]).reshape(-1, CH)
    zeros = jnp.zeros((np_rows // NS, d_out), jnp.float32)

    rows = e_pad // CH
    rpt1 = rows // NS
    seg1 = _segsum_kernel(
        np_rows, d_out, rpt1,
        src_base=lambda cid, sid: cid * rows + sid * rpt1,
        dst_base=lambda cid, sid: sid * rpt1,
    )
    agg = seg1(xh2, srcs1, dst2, zeros)
    a_x, a_h = agg[:n], agg[np_rows:np_rows + n]

    w_ru = jnp.concatenate([W_r, W_u], axis=1)
    wx_ru = w_ru[:d_in]
    wh_ru = w_ru[d_in:]
    b_ru = jnp.concatenate([b_r + r_bias, b_u + u_bias]).reshape(1, -1)
    wc_x = W_c[:d_in]
    wc_h = W_c[d_in:]
    b_cc = (b_c + c_bias).reshape(1, -1)

    br = 2000
    grid = (n // br,)
    row_spec = pl.BlockSpec((br, d_out), lambda i: (i, 0))
    full = lambda s: pl.BlockSpec(s, lambda i: (0, 0))
    h_, u, p = pl.pallas_call(
        _tc_gates,
        grid=grid,
        in_specs=[row_spec, row_spec, row_spec,
                  full((d_in, 2 * d_out)), full((d_out, 2 * d_out)),
                  full((1, 2 * d_out)), full((d_in, d_out))],
        out_specs=[row_spec, row_spec, row_spec],
        out_shape=[jax.ShapeDtypeStruct((n, d_out), jnp.float32)] * 3,
    )(a_x, a_h, h, wx_ru, wh_ru, b_ru, wc_x)

    src2b = jnp.concatenate([src, pad_src]).reshape(-1, CH)
    rpt2 = rows // (NC * NS)
    seg2 = _segsum_kernel(
        np_rows, d_out, rpt2,
        src_base=lambda cid, sid: (cid * NS + sid) * rpt2,
        dst_base=lambda cid, sid: (cid * NS + sid) * rpt2,
    )
    part = seg2(h_, src2b, dst2, zeros)
    b0, b1 = part[:n], part[np_rows:np_rows + n]

    new_h = pl.pallas_call(
        _tc_out,
        grid=grid,
        in_specs=[row_spec, row_spec, row_spec, row_spec, row_spec,
                  full((d_out, d_out)), full((1, d_out))],
        out_specs=row_spec,
        out_shape=jax.ShapeDtypeStruct((n, d_out), jnp.float32),
    )(p, b0, b1, h, u, wc_h, b_cc)
    return new_h


# speedup vs baseline: 10.4959x; 1.3795x over previous
import functools

import jax
import jax.numpy as jnp
from jax import lax
from jax.experimental import pallas as pl
from jax.experimental.pallas import tpu as pltpu
from jax.experimental.pallas import tpu_sc as plsc

NC = 2
NS = 16
CH = 64


def _segsum_kernel(np_rows, d, rows_per_tile, src_base, dst_base):
    npt = np_rows // NS
    mesh = plsc.VectorSubcoreMesh(core_axis_name="c", subcore_axis_name="s")

    @functools.partial(
        pl.kernel,
        out_type=jax.ShapeDtypeStruct((2 * np_rows, d), jnp.float32),
        mesh=mesh,
        scratch_types=[
            pltpu.VMEM((8, CH), jnp.int32),
            pltpu.VMEM((8, CH), jnp.int32),
            pltpu.VMEM((4, CH, d), jnp.float32),
            pltpu.VMEM_SHARED((np_rows, d), jnp.float32),
            pltpu.SemaphoreType.DMA,
            pltpu.SemaphoreType.DMA,
        ],
    )
    def seg(table_hbm, src2_hbm, dst2_hbm, zeros_hbm, out_hbm,
            src_v, dst_v, rows_v, accum, sem_g, sem_s):
        cid = lax.axis_index("c")
        sid = lax.axis_index("s")
        pltpu.sync_copy(zeros_hbm, accum.at[pl.ds(sid * npt, npt)])
        plsc.subcore_barrier()

        def body(j, carry):
            pltpu.sync_copy(
                src2_hbm.at[pl.ds(src_base(cid, sid) + j * 8, 8)], src_v)
            pltpu.sync_copy(
                dst2_hbm.at[pl.ds(dst_base(cid, sid) + j * 8, 8)], dst_v)
            gat = [pltpu.async_copy(table_hbm.at[src_v.at[g]],
                                    rows_v.at[g], sem_g)
                   for g in range(3)]
            sca = []
            for g in range(8):
                gat[g].wait()
                sca.append(pltpu.async_copy(rows_v.at[g % 4],
                                            accum.at[dst_v.at[g]],
                                            sem_s, add=True))
                nxt = g + 3
                if nxt < 8:
                    if nxt >= 4:
                        sca[nxt - 4].wait()
                    gat.append(pltpu.async_copy(table_hbm.at[src_v.at[nxt]],
                                                rows_v.at[nxt % 4], sem_g))
            for g in range(4, 8):
                sca[g].wait()
            return carry

        lax.fori_loop(0, rows_per_tile // 8, body, 0)
        plsc.subcore_barrier()
        pltpu.sync_copy(accum.at[pl.ds(sid * npt, npt)],
                        out_hbm.at[pl.ds(cid * np_rows + sid * npt, npt)])

    return seg


def _tc_gates(ax_ref, ah_ref, h_ref, wxru_ref, whru_ref, bru_ref, wcx_ref,
              hna_ref, u_ref, p_ref):
    ax = ax_ref[...]
    z = jnp.dot(ax, wxru_ref[...], preferred_element_type=jnp.float32)
    z = z + jnp.dot(ah_ref[...], whru_ref[...],
                    preferred_element_type=jnp.float32)
    ru = jax.nn.sigmoid(z + bru_ref[...])
    d = ax.shape[1]
    hna_ref[...] = ru[:, :d] * h_ref[...]
    u_ref[...] = ru[:, d:]
    p_ref[...] = jnp.dot(ax, wcx_ref[...], preferred_element_type=jnp.float32)


def _tc_out(p_ref, b0_ref, b1_ref, h_ref, u_ref, wch_ref, bc_ref, out_ref):
    b = b0_ref[...] + b1_ref[...]
    c = jax.nn.sigmoid(p_ref[...]
                       + jnp.dot(b, wch_ref[...],
                                 preferred_element_type=jnp.float32)
                       + bc_ref[...])
    u = u_ref[...]
    out_ref[...] = u * h_ref[...] + (1.0 - u) * c


def kernel(x, h, edge_index, W_r, b_r, W_u, b_u, W_c, b_c,
           r_bias, u_bias, c_bias):
    n, d_in = x.shape
    d_out = h.shape[1]
    e = edge_index.shape[1]
    src = edge_index[0]
    dst = edge_index[1]

    quant = CH * NS * NC * 8
    e_pad = -(-e // quant) * quant
    npad = e_pad - e
    np_rows = -(-n // (NS * 8)) * (NS * 8)
    pad_src = jnp.arange(npad, dtype=jnp.int32) % n
    pad_dst = n + jnp.arange(npad, dtype=jnp.int32) % (np_rows - n)

    xh2 = jnp.concatenate([x, h], axis=0)
    srcs1 = jnp.concatenate(
        [src, pad_src, src + n, pad_src + n]).reshape(-1, CH)
    dst2 = jnp.concatenate([dst, pad_dst]).reshape(-1, CH)
    zeros = jnp.zeros((np_rows // NS, d_out), jnp.float32)

    rows = e_pad // CH
    rpt1 = rows // NS
    seg1 = _segsum_kernel(
        np_rows, d_out, rpt1,
        src_base=lambda cid, sid: cid * rows + sid * rpt1,
        dst_base=lambda cid, sid: sid * rpt1,
    )
    agg = seg1(xh2, srcs1, dst2, zeros)
    a_x, a_h = agg[:n], agg[np_rows:np_rows + n]

    w_ru = jnp.concatenate([W_r, W_u], axis=1)
    wx_ru = w_ru[:d_in]
    wh_ru = w_ru[d_in:]
    b_ru = jnp.concatenate([b_r + r_bias, b_u + u_bias]).reshape(1, -1)
    wc_x = W_c[:d_in]
    wc_h = W_c[d_in:]
    b_cc = (b_c + c_bias).reshape(1, -1)

    br = 2000
    grid = (n // br,)
    row_spec = pl.BlockSpec((br, d_out), lambda i: (i, 0))
    full = lambda s: pl.BlockSpec(s, lambda i: (0, 0))
    h_, u, p = pl.pallas_call(
        _tc_gates,
        grid=grid,
        in_specs=[row_spec, row_spec, row_spec,
                  full((d_in, 2 * d_out)), full((d_out, 2 * d_out)),
                  full((1, 2 * d_out)), full((d_in, d_out))],
        out_specs=[row_spec, row_spec, row_spec],
        out_shape=[jax.ShapeDtypeStruct((n, d_out), jnp.float32)] * 3,
    )(a_x, a_h, h, wx_ru, wh_ru, b_ru, wc_x)

    src2b = jnp.concatenate([src, pad_src]).reshape(-1, CH)
    rpt2 = rows // (NC * NS)
    seg2 = _segsum_kernel(
        np_rows, d_out, rpt2,
        src_base=lambda cid, sid: (cid * NS + sid) * rpt2,
        dst_base=lambda cid, sid: (cid * NS + sid) * rpt2,
    )
    part = seg2(h_, src2b, dst2, zeros)
    b0, b1 = part[:n], part[np_rows:np_rows + n]

    new_h = pl.pallas_call(
        _tc_out,
        grid=grid,
        in_specs=[row_spec, row_spec, row_spec, row_spec, row_spec,
                  full((d_out, d_out)), full((1, d_out))],
        out_specs=row_spec,
        out_shape=jax.ShapeDtypeStruct((n, d_out), jnp.float32),
    )(p, b0, b1, h, u, wc_h, b_cc)
    return new_h


# speedup vs baseline: 10.6416x; 1.0139x over previous
import functools

import jax
import jax.numpy as jnp
from jax import lax
from jax.experimental import pallas as pl
from jax.experimental.pallas import tpu as pltpu
from jax.experimental.pallas import tpu_sc as plsc

NC = 2
NS = 16
CH = 64


def _segsum_kernel(np_rows, d, rows_per_tile, src_base, dst_base):
    npt = np_rows // NS
    mesh = plsc.VectorSubcoreMesh(core_axis_name="c", subcore_axis_name="s")

    NB = 5

    @functools.partial(
        pl.kernel,
        out_type=[jax.ShapeDtypeStruct((np_rows, d), jnp.float32),
                  jax.ShapeDtypeStruct((np_rows, d), jnp.float32)],
        mesh=mesh,
        scratch_types=[
            pltpu.VMEM((8, CH), jnp.int32),
            pltpu.VMEM((8, CH), jnp.int32),
            pltpu.VMEM((NB, CH, d), jnp.float32),
            pltpu.VMEM_SHARED((np_rows, d), jnp.float32),
            pltpu.SemaphoreType.DMA,
            pltpu.SemaphoreType.DMA,
        ],
    )
    def seg(table_hbm, src2_hbm, dst2_hbm, zeros_hbm, out0_hbm, out1_hbm,
            src_v, dst_v, rows_v, accum, sem_g, sem_s):
        cid = lax.axis_index("c")
        sid = lax.axis_index("s")
        pltpu.sync_copy(zeros_hbm, accum.at[pl.ds(sid * npt, npt)])
        plsc.subcore_barrier()

        def body(j, carry):
            pltpu.sync_copy(
                src2_hbm.at[pl.ds(src_base(cid, sid) + j * 8, 8)], src_v)
            pltpu.sync_copy(
                dst2_hbm.at[pl.ds(dst_base(cid, sid) + j * 8, 8)], dst_v)
            gat = [pltpu.async_copy(table_hbm.at[src_v.at[g]],
                                    rows_v.at[g], sem_g)
                   for g in range(NB - 1)]
            sca = []
            for g in range(8):
                gat[g].wait()
                sca.append(pltpu.async_copy(rows_v.at[g % NB],
                                            accum.at[dst_v.at[g]],
                                            sem_s, add=True))
                nxt = g + NB - 1
                if nxt < 8:
                    if nxt >= NB:
                        sca[nxt - NB].wait()
                    gat.append(pltpu.async_copy(table_hbm.at[src_v.at[nxt]],
                                                rows_v.at[nxt % NB], sem_g))
            for g in range(max(0, 8 - NB), 8):
                sca[g].wait()
            return carry

        lax.fori_loop(0, rows_per_tile // 8, body, 0)
        plsc.subcore_barrier()

        @pl.when(cid == 0)
        def _():
            pltpu.sync_copy(accum.at[pl.ds(sid * npt, npt)],
                            out0_hbm.at[pl.ds(sid * npt, npt)])

        @pl.when(cid == 1)
        def _():
            pltpu.sync_copy(accum.at[pl.ds(sid * npt, npt)],
                            out1_hbm.at[pl.ds(sid * npt, npt)])

    return seg


def _tc_gates(ax_ref, ah_ref, h_ref, wxru_ref, whru_ref, bru_ref, wcx_ref,
              hna_ref, u_ref, p_ref):
    ax = ax_ref[...]
    z = jnp.dot(ax, wxru_ref[...], preferred_element_type=jnp.float32)
    z = z + jnp.dot(ah_ref[...], whru_ref[...],
                    preferred_element_type=jnp.float32)
    ru = jax.nn.sigmoid(z + bru_ref[...])
    d = ax.shape[1]
    hna_ref[...] = ru[:, :d] * h_ref[...]
    u_ref[...] = ru[:, d:]
    p_ref[...] = jnp.dot(ax, wcx_ref[...], preferred_element_type=jnp.float32)


def _tc_out(p_ref, b0_ref, b1_ref, h_ref, u_ref, wch_ref, bc_ref, out_ref):
    b = b0_ref[...] + b1_ref[...]
    c = jax.nn.sigmoid(p_ref[...]
                       + jnp.dot(b, wch_ref[...],
                                 preferred_element_type=jnp.float32)
                       + bc_ref[...])
    u = u_ref[...]
    out_ref[...] = u * h_ref[...] + (1.0 - u) * c


def kernel(x, h, edge_index, W_r, b_r, W_u, b_u, W_c, b_c,
           r_bias, u_bias, c_bias):
    n, d_in = x.shape
    d_out = h.shape[1]
    e = edge_index.shape[1]
    src = edge_index[0]
    dst = edge_index[1]

    quant = CH * NS * NC * 8
    e_pad = -(-e // quant) * quant
    npad = e_pad - e
    np_rows = -(-n // (NS * 8)) * (NS * 8)
    pad_src = jnp.arange(npad, dtype=jnp.int32) % n
    pad_dst = n + jnp.arange(npad, dtype=jnp.int32) % (np_rows - n)

    xh2 = jnp.concatenate([x, h], axis=0)
    srcs1 = jnp.concatenate(
        [src, pad_src, src + n, pad_src + n]).reshape(-1, CH)
    dst2 = jnp.concatenate([dst, pad_dst]).reshape(-1, CH)
    zeros = jnp.zeros((np_rows // NS, d_out), jnp.float32)

    rows = e_pad // CH
    rpt1 = rows // NS
    seg1 = _segsum_kernel(
        np_rows, d_out, rpt1,
        src_base=lambda cid, sid: cid * rows + sid * rpt1,
        dst_base=lambda cid, sid: sid * rpt1,
    )
    a_x, a_h = seg1(xh2, srcs1, dst2, zeros)

    w_ru = jnp.concatenate([W_r, W_u], axis=1)
    wx_ru = w_ru[:d_in]
    wh_ru = w_ru[d_in:]
    b_ru = jnp.concatenate([b_r + r_bias, b_u + u_bias]).reshape(1, -1)
    wc_x = W_c[:d_in]
    wc_h = W_c[d_in:]
    b_cc = (b_c + c_bias).reshape(1, -1)

    br = 2000
    grid = (n // br,)
    row_spec = pl.BlockSpec((br, d_out), lambda i: (i, 0))
    full = lambda s: pl.BlockSpec(s, lambda i: (0, 0))
    h_, u, p = pl.pallas_call(
        _tc_gates,
        grid=grid,
        in_specs=[row_spec, row_spec, row_spec,
                  full((d_in, 2 * d_out)), full((d_out, 2 * d_out)),
                  full((1, 2 * d_out)), full((d_in, d_out))],
        out_specs=[row_spec, row_spec, row_spec],
        out_shape=[jax.ShapeDtypeStruct((n, d_out), jnp.float32)] * 3,
    )(a_x, a_h, h, wx_ru, wh_ru, b_ru, wc_x)

    src2b = jnp.concatenate([src, pad_src]).reshape(-1, CH)
    rpt2 = rows // (NC * NS)
    seg2 = _segsum_kernel(
        np_rows, d_out, rpt2,
        src_base=lambda cid, sid: (cid * NS + sid) * rpt2,
        dst_base=lambda cid, sid: (cid * NS + sid) * rpt2,
    )
    b0, b1 = seg2(h_, src2b, dst2, zeros)

    new_h = pl.pallas_call(
        _tc_out,
        grid=grid,
        in_specs=[row_spec, row_spec, row_spec, row_spec, row_spec,
                  full((d_out, d_out)), full((1, d_out))],
        out_specs=row_spec,
        out_shape=jax.ShapeDtypeStruct((n, d_out), jnp.float32),
    )(p, b0, b1, h, u, wc_h, b_cc)
    return new_h


# speedup vs baseline: 11.1606x; 1.0488x over previous
import functools

import jax
import jax.numpy as jnp
from jax import lax
from jax.experimental import pallas as pl
from jax.experimental.pallas import tpu as pltpu
from jax.experimental.pallas import tpu_sc as plsc

NC = 2
NS = 16
CH = 64


def _segsum_kernel(np_rows, d, rows_per_tile, src_base, dst_base):
    npt = np_rows // NS
    mesh = plsc.VectorSubcoreMesh(core_axis_name="c", subcore_axis_name="s")

    NB = 5

    @functools.partial(
        pl.kernel,
        out_type=[jax.ShapeDtypeStruct((np_rows, d), jnp.float32),
                  jax.ShapeDtypeStruct((np_rows, d), jnp.float32)],
        mesh=mesh,
        scratch_types=[
            pltpu.VMEM((8, CH), jnp.int32),
            pltpu.VMEM((8, CH), jnp.int32),
            pltpu.VMEM((NB, CH, d), jnp.float32),
            pltpu.VMEM_SHARED((np_rows, d), jnp.float32),
            pltpu.SemaphoreType.DMA,
            pltpu.SemaphoreType.DMA,
        ],
    )
    def seg(t0_hbm, t1_hbm, src2_hbm, dst2_hbm, zeros_hbm,
            out0_hbm, out1_hbm, src_v, dst_v, rows_v, accum, sem_g, sem_s):
        cid = lax.axis_index("c")
        sid = lax.axis_index("s")
        pltpu.sync_copy(zeros_hbm, accum.at[pl.ds(sid * npt, npt)])
        plsc.subcore_barrier()

        def run(table_hbm):
            def body(j, carry):
                pltpu.sync_copy(
                    src2_hbm.at[pl.ds(src_base(cid, sid) + j * 8, 8)], src_v)
                pltpu.sync_copy(
                    dst2_hbm.at[pl.ds(dst_base(cid, sid) + j * 8, 8)], dst_v)
                gat = [pltpu.async_copy(table_hbm.at[src_v.at[g]],
                                        rows_v.at[g], sem_g)
                       for g in range(NB - 1)]
                sca = []
                for g in range(8):
                    gat[g].wait()
                    sca.append(pltpu.async_copy(rows_v.at[g % NB],
                                                accum.at[dst_v.at[g]],
                                                sem_s, add=True))
                    nxt = g + NB - 1
                    if nxt < 8:
                        if nxt >= NB:
                            sca[nxt - NB].wait()
                        gat.append(
                            pltpu.async_copy(table_hbm.at[src_v.at[nxt]],
                                             rows_v.at[nxt % NB], sem_g))
                for g in range(max(0, 8 - NB), 8):
                    sca[g].wait()
                return carry

            lax.fori_loop(0, rows_per_tile // 8, body, 0)

        @pl.when(cid == 0)
        def _():
            run(t0_hbm)

        @pl.when(cid == 1)
        def _():
            run(t1_hbm)

        plsc.subcore_barrier()

        @pl.when(cid == 0)
        def _():
            pltpu.sync_copy(accum.at[pl.ds(sid * npt, npt)],
                            out0_hbm.at[pl.ds(sid * npt, npt)])

        @pl.when(cid == 1)
        def _():
            pltpu.sync_copy(accum.at[pl.ds(sid * npt, npt)],
                            out1_hbm.at[pl.ds(sid * npt, npt)])

    return seg


def _tc_gates(ax_ref, ah_ref, h_ref, wxru_ref, whru_ref, bru_ref,
              hna_ref, u_ref):
    z = jnp.dot(ax_ref[...], wxru_ref[...], preferred_element_type=jnp.float32)
    z = z + jnp.dot(ah_ref[...], whru_ref[...],
                    preferred_element_type=jnp.float32)
    ru = jax.nn.sigmoid(z + bru_ref[...])
    d = h_ref.shape[1]
    hna_ref[...] = ru[:, :d] * h_ref[...]
    u_ref[...] = ru[:, d:]


def _tc_out(ax_ref, b0_ref, b1_ref, h_ref, u_ref, wcx_ref, wch_ref, bc_ref,
            out_ref):
    b = b0_ref[...] + b1_ref[...]
    z = jnp.dot(ax_ref[...], wcx_ref[...], preferred_element_type=jnp.float32)
    z = z + jnp.dot(b, wch_ref[...], preferred_element_type=jnp.float32)
    c = jax.nn.sigmoid(z + bc_ref[...])
    u = u_ref[...]
    out_ref[...] = u * h_ref[...] + (1.0 - u) * c


def kernel(x, h, edge_index, W_r, b_r, W_u, b_u, W_c, b_c,
           r_bias, u_bias, c_bias):
    n, d_in = x.shape
    d_out = h.shape[1]
    e = edge_index.shape[1]
    src = edge_index[0]
    dst = edge_index[1]

    quant = CH * NS * NC * 8
    e_pad = -(-e // quant) * quant
    npad = e_pad - e
    np_rows = -(-n // (NS * 8)) * (NS * 8)
    pad_src = jnp.arange(npad, dtype=jnp.int32) % n
    pad_dst = n + jnp.arange(npad, dtype=jnp.int32) % (np_rows - n)

    srcs2 = jnp.concatenate([src, pad_src]).reshape(-1, CH)
    dst2 = jnp.concatenate([dst, pad_dst]).reshape(-1, CH)
    zeros = jnp.zeros((np_rows // NS, d_out), jnp.float32)

    rows = e_pad // CH
    rpt1 = rows // NS
    seg1 = _segsum_kernel(
        np_rows, d_out, rpt1,
        src_base=lambda cid, sid: sid * rpt1,
        dst_base=lambda cid, sid: sid * rpt1,
    )
    a_x, a_h = seg1(x, h, srcs2, dst2, zeros)

    w_ru = jnp.concatenate([W_r, W_u], axis=1)
    wx_ru = w_ru[:d_in]
    wh_ru = w_ru[d_in:]
    b_ru = jnp.concatenate([b_r + r_bias, b_u + u_bias]).reshape(1, -1)
    wc_x = W_c[:d_in]
    wc_h = W_c[d_in:]
    b_cc = (b_c + c_bias).reshape(1, -1)

    br = 2000
    grid = (n // br,)
    row_spec = pl.BlockSpec((br, d_out), lambda i: (i, 0))
    full = lambda s: pl.BlockSpec(s, lambda i: (0, 0))
    h_, u = pl.pallas_call(
        _tc_gates,
        grid=grid,
        in_specs=[row_spec, row_spec, row_spec,
                  full((d_in, 2 * d_out)), full((d_out, 2 * d_out)),
                  full((1, 2 * d_out))],
        out_specs=[row_spec, row_spec],
        out_shape=[jax.ShapeDtypeStruct((n, d_out), jnp.float32)] * 2,
    )(a_x, a_h, h, wx_ru, wh_ru, b_ru)

    rpt2 = rows // (NC * NS)
    seg2 = _segsum_kernel(
        np_rows, d_out, rpt2,
        src_base=lambda cid, sid: (cid * NS + sid) * rpt2,
        dst_base=lambda cid, sid: (cid * NS + sid) * rpt2,
    )
    b0, b1 = seg2(h_, h_, srcs2, dst2, zeros)

    new_h = pl.pallas_call(
        _tc_out,
        grid=grid,
        in_specs=[row_spec, row_spec, row_spec, row_spec, row_spec,
                  full((d_in, d_out)), full((d_out, d_out)),
                  full((1, d_out))],
        out_specs=row_spec,
        out_shape=jax.ShapeDtypeStruct((n, d_out), jnp.float32),
    )(a_x, b0, b1, h, u, wc_x, wc_h, b_cc)
    return new_h


# speedup vs baseline: 12.8974x; 1.1556x over previous
import functools

import jax
import jax.numpy as jnp
from jax import lax
from jax.experimental import pallas as pl
from jax.experimental.pallas import tpu as pltpu
from jax.experimental.pallas import tpu_sc as plsc

NC = 2
NS = 16
CH = 64


def _segsum_kernel(np_rows, d, rows_per_tile, src_base, dst_base):
    npt = np_rows // NS
    mesh = plsc.VectorSubcoreMesh(core_axis_name="c", subcore_axis_name="s")

    NB = 5

    @functools.partial(
        pl.kernel,
        out_type=[jax.ShapeDtypeStruct((np_rows, d), jnp.float32),
                  jax.ShapeDtypeStruct((np_rows, d), jnp.float32)],
        mesh=mesh,
        scratch_types=[
            pltpu.VMEM((2, 8, CH), jnp.int32),
            pltpu.VMEM((2, 8, CH), jnp.int32),
            pltpu.VMEM((NB, CH, d), jnp.float32),
            pltpu.VMEM_SHARED((np_rows, d), jnp.float32),
            pltpu.SemaphoreType.DMA,
            pltpu.SemaphoreType.DMA,
            pltpu.SemaphoreType.DMA,
        ],
    )
    def seg(t0_hbm, t1_hbm, src2_hbm, dst2_hbm, zeros_hbm,
            out0_hbm, out1_hbm, src_v, dst_v, rows_v, accum,
            sem_g, sem_s, sem_i):
        cid = lax.axis_index("c")
        sid = lax.axis_index("s")
        pltpu.sync_copy(zeros_hbm, accum.at[pl.ds(sid * npt, npt)])
        plsc.subcore_barrier()

        ngroups = rows_per_tile // 8
        src0 = src_base(cid, sid)
        dst0 = dst_base(cid, sid)

        def run(table_hbm):
            pltpu.sync_copy(src2_hbm.at[pl.ds(src0, 8)], src_v.at[0])
            pltpu.sync_copy(dst2_hbm.at[pl.ds(dst0, 8)], dst_v.at[0])

            def body(j, carry):
                cur = lax.rem(j, 2)
                jn = jnp.minimum(j + 1, ngroups - 1)
                st_s = pltpu.async_copy(
                    src2_hbm.at[pl.ds(src0 + jn * 8, 8)],
                    src_v.at[1 - cur], sem_i)
                st_d = pltpu.async_copy(
                    dst2_hbm.at[pl.ds(dst0 + jn * 8, 8)],
                    dst_v.at[1 - cur], sem_i)
                gat = [pltpu.async_copy(table_hbm.at[src_v.at[cur, g]],
                                        rows_v.at[g], sem_g)
                       for g in range(NB - 1)]
                sca = []
                for g in range(8):
                    gat[g].wait()
                    sca.append(pltpu.async_copy(rows_v.at[g % NB],
                                                accum.at[dst_v.at[cur, g]],
                                                sem_s, add=True))
                    nxt = g + NB - 1
                    if nxt < 8:
                        if nxt >= NB:
                            sca[nxt - NB].wait()
                        gat.append(
                            pltpu.async_copy(table_hbm.at[src_v.at[cur, nxt]],
                                             rows_v.at[nxt % NB], sem_g))
                for g in range(max(0, 8 - NB), 8):
                    sca[g].wait()
                st_s.wait()
                st_d.wait()
                return carry

            lax.fori_loop(0, ngroups, body, 0)

        @pl.when(cid == 0)
        def _():
            run(t0_hbm)

        @pl.when(cid == 1)
        def _():
            run(t1_hbm)

        plsc.subcore_barrier()

        @pl.when(cid == 0)
        def _():
            pltpu.sync_copy(accum.at[pl.ds(sid * npt, npt)],
                            out0_hbm.at[pl.ds(sid * npt, npt)])

        @pl.when(cid == 1)
        def _():
            pltpu.sync_copy(accum.at[pl.ds(sid * npt, npt)],
                            out1_hbm.at[pl.ds(sid * npt, npt)])

    return seg


def _tc_gates(ax_ref, ah_ref, h_ref, wxru_ref, whru_ref, bru_ref,
              hna_ref, u_ref):
    z = jnp.dot(ax_ref[...], wxru_ref[...], preferred_element_type=jnp.float32)
    z = z + jnp.dot(ah_ref[...], whru_ref[...],
                    preferred_element_type=jnp.float32)
    ru = jax.nn.sigmoid(z + bru_ref[...])
    d = h_ref.shape[1]
    hna_ref[...] = ru[:, :d] * h_ref[...]
    u_ref[...] = ru[:, d:]


def _tc_out(ax_ref, b0_ref, b1_ref, h_ref, u_ref, wcx_ref, wch_ref, bc_ref,
            out_ref):
    b = b0_ref[...] + b1_ref[...]
    z = jnp.dot(ax_ref[...], wcx_ref[...], preferred_element_type=jnp.float32)
    z = z + jnp.dot(b, wch_ref[...], preferred_element_type=jnp.float32)
    c = jax.nn.sigmoid(z + bc_ref[...])
    u = u_ref[...]
    out_ref[...] = u * h_ref[...] + (1.0 - u) * c


def kernel(x, h, edge_index, W_r, b_r, W_u, b_u, W_c, b_c,
           r_bias, u_bias, c_bias):
    n, d_in = x.shape
    d_out = h.shape[1]
    e = edge_index.shape[1]
    src = edge_index[0]
    dst = edge_index[1]

    quant = CH * NS * NC * 8
    e_pad = -(-e // quant) * quant
    npad = e_pad - e
    np_rows = -(-n // (NS * 8)) * (NS * 8)
    pad_src = jnp.arange(npad, dtype=jnp.int32) % n
    pad_dst = n + jnp.arange(npad, dtype=jnp.int32) % (np_rows - n)

    srcs2 = jnp.concatenate([src, pad_src]).reshape(-1, CH)
    dst2 = jnp.concatenate([dst, pad_dst]).reshape(-1, CH)
    zeros = jnp.zeros((np_rows // NS, d_out), jnp.float32)

    rows = e_pad // CH
    rpt1 = rows // NS
    seg1 = _segsum_kernel(
        np_rows, d_out, rpt1,
        src_base=lambda cid, sid: sid * rpt1,
        dst_base=lambda cid, sid: sid * rpt1,
    )
    a_x, a_h = seg1(x, h, srcs2, dst2, zeros)

    w_ru = jnp.concatenate([W_r, W_u], axis=1)
    wx_ru = w_ru[:d_in]
    wh_ru = w_ru[d_in:]
    b_ru = jnp.concatenate([b_r + r_bias, b_u + u_bias]).reshape(1, -1)
    wc_x = W_c[:d_in]
    wc_h = W_c[d_in:]
    b_cc = (b_c + c_bias).reshape(1, -1)

    br = 2000
    grid = (n // br,)
    row_spec = pl.BlockSpec((br, d_out), lambda i: (i, 0))
    full = lambda s: pl.BlockSpec(s, lambda i: (0, 0))
    h_, u = pl.pallas_call(
        _tc_gates,
        grid=grid,
        in_specs=[row_spec, row_spec, row_spec,
                  full((d_in, 2 * d_out)), full((d_out, 2 * d_out)),
                  full((1, 2 * d_out))],
        out_specs=[row_spec, row_spec],
        out_shape=[jax.ShapeDtypeStruct((n, d_out), jnp.float32)] * 2,
    )(a_x, a_h, h, wx_ru, wh_ru, b_ru)

    rpt2 = rows // (NC * NS)
    seg2 = _segsum_kernel(
        np_rows, d_out, rpt2,
        src_base=lambda cid, sid: (cid * NS + sid) * rpt2,
        dst_base=lambda cid, sid: (cid * NS + sid) * rpt2,
    )
    b0, b1 = seg2(h_, h_, srcs2, dst2, zeros)

    new_h = pl.pallas_call(
        _tc_out,
        grid=grid,
        in_specs=[row_spec, row_spec, row_spec, row_spec, row_spec,
                  full((d_in, d_out)), full((d_out, d_out)),
                  full((1, d_out))],
        out_specs=row_spec,
        out_shape=jax.ShapeDtypeStruct((n, d_out), jnp.float32),
    )(a_x, b0, b1, h, u, wc_x, wc_h, b_cc)
    return new_h
